# baseline (speedup 1.0000x reference)
import functools

import jax
import jax.numpy as jnp
from jax import lax
from jax.experimental import pallas as pl
from jax.experimental.pallas import tpu as pltpu
from jax.experimental.pallas import tpu_sc as plsc

PADDING_IDX = 1
LANES = 16

_NC = 2
_NS = 16
_NW = _NC * _NS


def _make_sc_bcast(bsz, seq, d):
    cols_pw = seq // _NW
    assert seq % _NW == 0 and cols_pw % LANES == 0 and d % LANES == 0
    ch = 32
    nch = cols_pw // ch
    assert cols_pw % ch == 0
    gpb = cols_pw // LANES
    ngroups = bsz * gpb
    mesh = plsc.VectorSubcoreMesh(core_axis_name="c", subcore_axis_name="s")

    @functools.partial(
        pl.kernel,
        mesh=mesh,
        compiler_params=pltpu.CompilerParams(needs_layout_passes=False),
        out_type=jax.ShapeDtypeStruct((bsz * seq * d,), jnp.float32),
        scratch_types=[
            pltpu.VMEM((bsz * cols_pw,), jnp.int32),
            pltpu.VMEM((d,), jnp.float32),
            pltpu.VMEM((ch * d,), jnp.float32),
            pltpu.VMEM((ch * d,), jnp.float32),
            pltpu.SemaphoreType.DMA,
            pltpu.SemaphoreType.DMA,
            pltpu.SemaphoreType.DMA,
            pltpu.SemaphoreType.DMA,
        ],
    )
    def body(tok_hbm, w_hbm, out_hbm, tok_v, zrow_v, buf0, buf1,
             sr0, sr1, sw0, sw1):
        wid = lax.axis_index("s") * _NC + lax.axis_index("c")
        col0 = wid * cols_pw
        bufs, srs, sws = (buf0, buf1), (sr0, sr1), (sw0, sw1)

        for b in range(bsz):
            pltpu.sync_copy(tok_hbm.at[pl.ds(b * seq + col0, cols_pw)],
                            tok_v.at[pl.ds(b * cols_pw, cols_pw)])

        zeros16 = jnp.zeros((LANES,), jnp.float32)

        def zbody(i, carry):
            zrow_v[pl.ds(i * LANES, LANES)] = zeros16
            return carry

        lax.fori_loop(0, d // LANES, zbody, 0)

        writes = {}

        def start_read(c):
            bi = c % 2
            return pltpu.async_copy(
                w_hbm.at[pl.ds((col0 + (PADDING_IDX + 1) + c * ch) * d,
                               ch * d)],
                bufs[bi], srs[bi])

        pending_read = start_read(0)
        for c in range(nch):
            bi = c % 2
            pending_read.wait()
            if c + 1 < nch:
                for w in writes.pop(1 - bi, ()):
                    w.wait()
                pending_read = start_read(c + 1)
            writes[bi] = [
                pltpu.async_copy(
                    bufs[bi],
                    out_hbm.at[pl.ds((b * seq + col0 + c * ch) * d, ch * d)],
                    sws[bi])
                for b in range(bsz)
            ]
        for bi in (0, 1):
            for w in writes.pop(bi, ()):
                w.wait()

        def fix_group(g, carry):
            t = tok_v[pl.ds(g * LANES, LANES)]
            pad = t == PADDING_IDX
            cnt = plsc.all_reduce_population_count(pad)

            @pl.when(cnt[0] > 0)
            def _():
                b = g // gpb
                cbase = (g % gpb) * LANES
                for l in range(LANES):
                    @pl.when(t[l] == PADDING_IDX)
                    def _():
                        row = b * seq + col0 + cbase + l
                        pltpu.sync_copy(zrow_v, out_hbm.at[pl.ds(row * d, d)])

            return carry

        lax.fori_loop(0, ngroups, fix_group, 0)

    return body


def kernel(input, weights):
    bsz, seq = input.shape
    _, d = weights.shape
    lookup = _make_sc_bcast(bsz, seq, d)
    out = lookup(input.reshape(-1), weights.reshape(-1))
    return out.reshape(bsz, seq, d)


# speedup vs baseline: 3.0849x; 3.0849x over previous
import functools

import jax
import jax.numpy as jnp
from jax import lax
from jax.experimental import pallas as pl
from jax.experimental.pallas import tpu as pltpu
from jax.experimental.pallas import tpu_sc as plsc

PADDING_IDX = 1
LANES = 16

_NC = 2
_NS = 16
_NW = _NC * _NS


def _make_sc_bcast(bsz, seq, d):
    cols_pw = seq // _NW
    assert seq % _NW == 0 and cols_pw % LANES == 0 and d % LANES == 0
    ch = 32
    nch = cols_pw // ch
    assert cols_pw % ch == 0
    gpb = cols_pw // LANES
    ngroups = bsz * gpb
    mesh = plsc.VectorSubcoreMesh(core_axis_name="c", subcore_axis_name="s")

    @functools.partial(
        pl.kernel,
        mesh=mesh,
        compiler_params=pltpu.CompilerParams(needs_layout_passes=False),
        out_type=jax.ShapeDtypeStruct((bsz * seq, d), jnp.float32),
        scratch_types=[
            pltpu.VMEM((bsz * cols_pw,), jnp.int32),
            pltpu.VMEM((ch,), jnp.int32),
            pltpu.VMEM((ch,), jnp.int32),
            pltpu.VMEM((LANES,), jnp.int32),
            pltpu.VMEM((ch, d), jnp.float32),
            pltpu.VMEM((ch, d), jnp.float32),
            pltpu.VMEM((LANES, d), jnp.float32),
            pltpu.SemaphoreType.DMA,
            pltpu.SemaphoreType.DMA,
            pltpu.SemaphoreType.DMA,
            pltpu.SemaphoreType.DMA,
            pltpu.SemaphoreType.DMA,
        ],
    )
    def body(tok_hbm, w_hbm, out_hbm, tok_v, idx0, idx1, idxg, buf0, buf1,
             gbuf, sr0, sr1, sw0, sw1, sg):
        wid = lax.axis_index("s") * _NC + lax.axis_index("c")
        col0 = wid * cols_pw
        bufs, idxs, srs, sws = (buf0, buf1), (idx0, idx1), (sr0, sr1), (sw0, sw1)
        lane = lax.broadcasted_iota(jnp.int32, (LANES,), 0)

        for b in range(bsz):
            pltpu.sync_copy(tok_hbm.at[pl.ds(b * seq + col0, cols_pw)],
                            tok_v.at[pl.ds(b * cols_pw, cols_pw)])

        writes = {}

        def start_read(c):
            bi = c % 2
            for g in range(ch // LANES):
                idxs[bi][pl.ds(g * LANES, LANES)] = (
                    lane + (col0 + (PADDING_IDX + 1) + c * ch + g * LANES))
            return pltpu.async_copy(w_hbm.at[idxs[bi]], bufs[bi], srs[bi])

        pending_read = start_read(0)
        for c in range(nch):
            bi = c % 2
            pending_read.wait()
            if c + 1 < nch:
                for w in writes.pop(1 - bi, ()):
                    w.wait()
                pending_read = start_read(c + 1)
            writes[bi] = [
                pltpu.async_copy(
                    bufs[bi],
                    out_hbm.at[pl.ds(b * seq + col0 + c * ch, ch)],
                    sws[bi])
                for b in range(bsz)
            ]
        for bi in (0, 1):
            for w in writes.pop(bi, ()):
                w.wait()

        def fix_group(g, carry):
            t = tok_v[pl.ds(g * LANES, LANES)]
            pad = t == PADDING_IDX
            cnt = plsc.all_reduce_population_count(pad)

            @pl.when(cnt[0] > 0)
            def _():
                b = g // gpb
                cbase = (g % gpb) * LANES
                col = lane + (col0 + cbase)
                pos = jnp.where(pad, PADDING_IDX, col + (PADDING_IDX + 1))
                idxg[...] = pos
                pltpu.async_copy(w_hbm.at[idxg], gbuf, sg).wait()
                pltpu.sync_copy(
                    gbuf, out_hbm.at[pl.ds(b * seq + col0 + cbase, LANES)])

            return carry

        lax.fori_loop(0, ngroups, fix_group, 0)

    return body


def kernel(input, weights):
    bsz, seq = input.shape
    _, d = weights.shape
    lookup = _make_sc_bcast(bsz, seq, d)
    out = lookup(input.reshape(-1), weights)
    return out.reshape(bsz, seq, d)
